# baseline (speedup 1.0000x reference)
import jax
import jax.numpy as jnp
from jax import lax
from jax.experimental import pallas as pl
from jax.experimental.pallas import tpu as pltpu
from jax.experimental.pallas import tpu_sc as plsc

N = 10000
NP = 10240
D = 128
G = 128
E = 320000
NW = 32
CHUNK = 128
NCHUNK = 80
HALF = NCHUNK // 2
HGROUP = HALF // 2
EP = NW * NCHUNK * CHUNK
RPS = NP // 16

_mesh = plsc.VectorSubcoreMesh(
    core_axis_name="c", subcore_axis_name="s", num_cores=2, num_subcores=16
)


def _deg_body(dst_hbm, out_hbm, dstv, hist):
    c = lax.axis_index("c")
    s = lax.axis_index("s")
    wid = c * 16 + s
    pltpu.sync_copy(dst_hbm.at[wid], dstv)

    def zero_body(i, carry):
        hist[pl.ds(i * 16, 16)] = jnp.zeros((16,), jnp.float32)
        return carry

    lax.fori_loop(0, NP // 16, zero_body, 0)

    ones = jnp.ones((16,), jnp.float32)

    def hist_body(k, carry):
        kc = k // (CHUNK // 16)
        kj = k % (CHUNK // 16)
        idx = dstv[kc, pl.ds(kj * 16, 16)]
        plsc.addupdate_scatter(hist, [idx], ones)
        return carry

    lax.fori_loop(0, NCHUNK * (CHUNK // 16), hist_body, 0)
    pltpu.sync_copy(hist, out_hbm.at[wid])


_sc_params = pltpu.CompilerParams(needs_layout_passes=False)

_deg_call = pl.kernel(
    _deg_body,
    out_type=jax.ShapeDtypeStruct((NW, NP), jnp.float32),
    mesh=_mesh,
    compiler_params=_sc_params,
    scratch_types=[
        pltpu.VMEM((NCHUNK, CHUNK), jnp.int32),
        pltpu.VMEM((NP,), jnp.float32),
    ],
)


def _layer_body(
    hp_hbm, src_hbm, dst_hbm, out_hbm, srcv, dstv, rows0, rows1, acc,
    gsem0, gsem1, ssem0, ssem1,
):
    c = lax.axis_index("c")
    s = lax.axis_index("s")
    wid = c * 16 + s

    def zero_body(i, carry):
        rows0[i // (D // 16), pl.ds((i % (D // 16)) * 16, 16)] = jnp.zeros(
            (16,), jnp.float32
        )
        return carry

    lax.fori_loop(0, CHUNK * (D // 16), zero_body, 0)
    for r in range(RPS // CHUNK):
        pltpu.sync_copy(rows0, acc.at[pl.ds(s * RPS + r * CHUNK, CHUNK)])
    plsc.subcore_barrier()

    for phase in range(NCHUNK // HALF):
        base = phase * HALF
        pltpu.sync_copy(src_hbm.at[wid, pl.ds(base, HALF)], srcv)
        pltpu.sync_copy(dst_hbm.at[wid, pl.ds(base, HALF)], dstv)
        pltpu.async_copy(hp_hbm.at[srcv.at[0]], rows0, gsem0)
        pltpu.async_copy(hp_hbm.at[srcv.at[1]], rows1, gsem1)

        def group_body(g, carry):
            j0 = 2 * g
            j1 = 2 * g + 1
            pltpu.make_async_copy(hp_hbm.at[srcv.at[j0]], rows0, gsem0).wait()
            pltpu.async_copy(rows0, acc.at[dstv.at[j0]], ssem0, add=True)
            pltpu.make_async_copy(hp_hbm.at[srcv.at[j1]], rows1, gsem1).wait()
            pltpu.async_copy(rows1, acc.at[dstv.at[j1]], ssem1, add=True)

            @pl.when(g + 1 < HGROUP)
            def _():
                pltpu.make_async_copy(
                    rows0, acc.at[dstv.at[j0]], ssem0
                ).wait()
                pltpu.async_copy(hp_hbm.at[srcv.at[j0 + 2]], rows0, gsem0)
                pltpu.make_async_copy(
                    rows1, acc.at[dstv.at[j1]], ssem1
                ).wait()
                pltpu.async_copy(hp_hbm.at[srcv.at[j1 + 2]], rows1, gsem1)

            return carry

        lax.fori_loop(0, HGROUP, group_body, 0)
        pltpu.make_async_copy(rows0, acc.at[dstv.at[HALF - 2]], ssem0).wait()
        pltpu.make_async_copy(rows1, acc.at[dstv.at[HALF - 1]], ssem1).wait()

    plsc.subcore_barrier()
    pltpu.sync_copy(
        acc.at[pl.ds(s * RPS, RPS)], out_hbm.at[pl.ds(c * NP + s * RPS, RPS)]
    )


_layer_call = pl.kernel(
    _layer_body,
    out_type=jax.ShapeDtypeStruct((2 * NP, D), jnp.float32),
    mesh=_mesh,
    compiler_params=_sc_params,
    scratch_types=[
        pltpu.VMEM((HALF, CHUNK), jnp.int32),
        pltpu.VMEM((HALF, CHUNK), jnp.int32),
        pltpu.VMEM((CHUNK, D), jnp.float32),
        pltpu.VMEM((CHUNK, D), jnp.float32),
        pltpu.VMEM_SHARED((NP, D), jnp.float32),
        pltpu.SemaphoreType.DMA,
        pltpu.SemaphoreType.DMA,
        pltpu.SemaphoreType.DMA,
        pltpu.SemaphoreType.DMA,
    ],
)


def _prep_body(partials_ref, x_ref, w1_ref, dis_ref, hp_ref):
    deg = 1.0 + jnp.sum(partials_ref[...], axis=1, keepdims=True)
    row = lax.broadcasted_iota(jnp.int32, (NP, 1), 0)
    dis = jnp.where(row < N, lax.rsqrt(deg), 0.0)
    dis_ref[...] = dis
    hp_ref[...] = dis * jnp.dot(
        x_ref[...], w1_ref[...], preferred_element_type=jnp.float32
    )


_prep_call = pl.pallas_call(
    _prep_body,
    out_shape=(
        jax.ShapeDtypeStruct((NP, 1), jnp.float32),
        jax.ShapeDtypeStruct((NP, D), jnp.float32),
    ),
)


def _mid_body(acc_ref, hp_ref, dis_ref, b_ref, w_ref, out_ref):
    tot = acc_ref[0:NP, :] + acc_ref[NP : 2 * NP, :] + hp_ref[...]
    t = jnp.tanh(dis_ref[...] * tot + b_ref[...])
    out_ref[...] = dis_ref[...] * jnp.dot(
        t, w_ref[...], preferred_element_type=jnp.float32
    )


_mid_call = pl.pallas_call(
    _mid_body,
    out_shape=jax.ShapeDtypeStruct((NP, D), jnp.float32),
)


def _final_body(
    acc_ref, hp_ref, dis_ref, b_ref, batch_ref, wlin_ref, blin_ref, out_ref, emb_ref
):
    tot = acc_ref[0:NP, :] + acc_ref[NP : 2 * NP, :] + hp_ref[...]
    t = jnp.tanh(dis_ref[...] * tot + b_ref[...])
    gids = lax.broadcasted_iota(jnp.int32, (G, NP), 0)
    onehot = (batch_ref[...] == gids).astype(jnp.float32)
    sums = jnp.dot(onehot, t, preferred_element_type=jnp.float32)
    counts = jnp.sum(onehot, axis=1, keepdims=True)
    emb = sums / jnp.maximum(counts, 1.0)
    emb_ref[...] = emb
    out_ref[...] = (
        jnp.dot(emb, wlin_ref[...], preferred_element_type=jnp.float32)
        + blin_ref[...]
    )


_final_call = pl.pallas_call(
    _final_body,
    out_shape=(
        jax.ShapeDtypeStruct((G, 10), jnp.float32),
        jax.ShapeDtypeStruct((G, D), jnp.float32),
    ),
)


def kernel(x, edge_index, batch, W1, b1, W2, b2, W3, b3, W4, b4, W5, b5, Wlin, blin):
    src = edge_index[0].astype(jnp.int32)
    dst = edge_index[1].astype(jnp.int32)
    ppt = (EP - E) // NW
    padv = (N + jnp.arange(EP - E, dtype=jnp.int32) % (NP - N)).reshape(NW, ppt)
    src3 = jnp.concatenate(
        [src.reshape(NW, E // NW), padv], axis=1
    ).reshape(NW, NCHUNK, CHUNK)
    dst3 = jnp.concatenate(
        [dst.reshape(NW, E // NW), padv], axis=1
    ).reshape(NW, NCHUNK, CHUNK)
    x_pad = jnp.concatenate([x, jnp.zeros((NP - N, D), x.dtype)], axis=0)
    batch_pad = jnp.concatenate(
        [batch.astype(jnp.int32), jnp.full((NP - N,), G, jnp.int32)]
    ).reshape(1, NP)

    partials = _deg_call(dst3)
    dis, hp = _prep_call(partials.T, x_pad, W1)

    for W_next, b_prev in ((W2, b1), (W3, b2), (W4, b3), (W5, b4)):
        acc = _layer_call(hp, src3, dst3)
        hp = _mid_call(acc, hp, dis, b_prev.reshape(1, D), W_next)

    acc = _layer_call(hp, src3, dst3)
    out, emb = _final_call(
        acc, hp, dis, b5.reshape(1, D), batch_pad, Wlin, blin.reshape(1, -1)
    )
    return (out, emb)


# speedup vs baseline: 1.2589x; 1.2589x over previous
import jax
import jax.numpy as jnp
from jax import lax
from jax.experimental import pallas as pl
from jax.experimental.pallas import tpu as pltpu
from jax.experimental.pallas import tpu_sc as plsc

N = 10000
NP = 10240
D = 128
G = 128
E = 320000
NW = 32
CHUNK = 128
NCHUNK = 80
HALF = NCHUNK // 2
HGROUP = HALF // 2
EP = NW * NCHUNK * CHUNK
RPS = NP // 16

_mesh = plsc.VectorSubcoreMesh(
    core_axis_name="c", subcore_axis_name="s", num_cores=2, num_subcores=16
)


def _deg_body(dst_hbm, out_hbm, dstv, hist):
    c = lax.axis_index("c")
    s = lax.axis_index("s")
    wid = c * 16 + s
    pltpu.sync_copy(dst_hbm.at[wid], dstv)

    def zero_body(i, carry):
        hist[pl.ds(i * 16, 16)] = jnp.zeros((16,), jnp.float32)
        return carry

    lax.fori_loop(0, NP // 16, zero_body, 0)

    ones = jnp.ones((16,), jnp.float32)

    def hist_body(k, carry):
        kc = k // (CHUNK // 16)
        kj = k % (CHUNK // 16)
        idx = dstv[kc, pl.ds(kj * 16, 16)]
        plsc.addupdate_scatter(hist, [idx], ones)
        return carry

    lax.fori_loop(0, NCHUNK * (CHUNK // 16), hist_body, 0)
    pltpu.sync_copy(hist, out_hbm.at[wid])


_sc_params = pltpu.CompilerParams(needs_layout_passes=False)

_deg_call = pl.kernel(
    _deg_body,
    out_type=jax.ShapeDtypeStruct((NW, NP), jnp.float32),
    mesh=_mesh,
    compiler_params=_sc_params,
    scratch_types=[
        pltpu.VMEM((NCHUNK, CHUNK), jnp.int32),
        pltpu.VMEM((NP,), jnp.float32),
    ],
)


def _layer_body(
    hp_hbm, src_hbm, dst_hbm, out_hbm, srcv, dstv, rows0, rows1, acc,
    gsem0, gsem1,
):
    c = lax.axis_index("c")
    s = lax.axis_index("s")
    wid = c * 16 + s

    def zero_body(i, carry):
        rows0[i // (D // 16), pl.ds((i % (D // 16)) * 16, 16)] = jnp.zeros(
            (16,), jnp.float32
        )
        return carry

    lax.fori_loop(0, CHUNK * (D // 16), zero_body, 0)
    for r in range(RPS // CHUNK):
        pltpu.sync_copy(rows0, acc.at[pl.ds(s * RPS + r * CHUNK, CHUNK)])
    plsc.subcore_barrier()

    for phase in range(NCHUNK // HALF):
        base = phase * HALF
        pltpu.sync_copy(src_hbm.at[wid, pl.ds(base, HALF)], srcv)
        pltpu.sync_copy(dst_hbm.at[wid, pl.ds(base, HALF)], dstv)
        pltpu.async_copy(hp_hbm.at[srcv.at[0]], rows0, gsem0)

        def group_body(g, carry):
            j0 = 2 * g
            j1 = 2 * g + 1
            pltpu.async_copy(hp_hbm.at[srcv.at[j1]], rows1, gsem1)
            pltpu.make_async_copy(hp_hbm.at[srcv.at[j0]], rows0, gsem0).wait()
            pltpu.sync_copy(rows0, acc.at[dstv.at[j0]], add=True)

            @pl.when(g + 1 < HGROUP)
            def _():
                pltpu.async_copy(hp_hbm.at[srcv.at[j0 + 2]], rows0, gsem0)

            pltpu.make_async_copy(hp_hbm.at[srcv.at[j1]], rows1, gsem1).wait()
            pltpu.sync_copy(rows1, acc.at[dstv.at[j1]], add=True)
            return carry

        lax.fori_loop(0, HGROUP, group_body, 0)

    plsc.subcore_barrier()
    pltpu.sync_copy(
        acc.at[pl.ds(s * RPS, RPS)], out_hbm.at[pl.ds(c * NP + s * RPS, RPS)]
    )


_layer_call = pl.kernel(
    _layer_body,
    out_type=jax.ShapeDtypeStruct((2 * NP, D), jnp.float32),
    mesh=_mesh,
    compiler_params=_sc_params,
    scratch_types=[
        pltpu.VMEM((HALF, CHUNK), jnp.int32),
        pltpu.VMEM((HALF, CHUNK), jnp.int32),
        pltpu.VMEM((CHUNK, D), jnp.float32),
        pltpu.VMEM((CHUNK, D), jnp.float32),
        pltpu.VMEM_SHARED((NP, D), jnp.float32),
        pltpu.SemaphoreType.DMA,
        pltpu.SemaphoreType.DMA,
    ],
)


BLK = NP // 8


def _prep_body(partials_ref, x_ref, w1_ref, dis_ref, hp_ref):
    i = pl.program_id(0)
    deg = 1.0 + jnp.sum(partials_ref[...], axis=1, keepdims=True)
    row = i * BLK + lax.broadcasted_iota(jnp.int32, (BLK, 1), 0)
    dis = jnp.where(row < N, lax.rsqrt(deg), 0.0)
    dis_ref[...] = dis
    hp_ref[...] = dis * jnp.dot(
        x_ref[...], w1_ref[...], preferred_element_type=jnp.float32
    )


_prep_call = pl.pallas_call(
    _prep_body,
    grid=(NP // BLK,),
    in_specs=[
        pl.BlockSpec((BLK, NW), lambda i: (i, 0)),
        pl.BlockSpec((BLK, D), lambda i: (i, 0)),
        pl.BlockSpec((D, D), lambda i: (0, 0)),
    ],
    out_specs=(
        pl.BlockSpec((BLK, 1), lambda i: (i, 0)),
        pl.BlockSpec((BLK, D), lambda i: (i, 0)),
    ),
    out_shape=(
        jax.ShapeDtypeStruct((NP, 1), jnp.float32),
        jax.ShapeDtypeStruct((NP, D), jnp.float32),
    ),
)


def _mid_body(acc_ref, hp_ref, dis_ref, b_ref, w_ref, out_ref):
    tot = acc_ref[0] + acc_ref[1] + hp_ref[...]
    t = jnp.tanh(dis_ref[...] * tot + b_ref[...])
    out_ref[...] = dis_ref[...] * jnp.dot(
        t, w_ref[...], preferred_element_type=jnp.float32
    )


_mid_call = pl.pallas_call(
    _mid_body,
    grid=(NP // BLK,),
    in_specs=[
        pl.BlockSpec((2, BLK, D), lambda i: (0, i, 0)),
        pl.BlockSpec((BLK, D), lambda i: (i, 0)),
        pl.BlockSpec((BLK, 1), lambda i: (i, 0)),
        pl.BlockSpec((1, D), lambda i: (0, 0)),
        pl.BlockSpec((D, D), lambda i: (0, 0)),
    ],
    out_specs=pl.BlockSpec((BLK, D), lambda i: (i, 0)),
    out_shape=jax.ShapeDtypeStruct((NP, D), jnp.float32),
)


def _final_body(
    acc_ref, hp_ref, dis_ref, b_ref, batch_ref, wlin_ref, blin_ref, out_ref, emb_ref
):
    tot = acc_ref[0] + acc_ref[1] + hp_ref[...]
    t = jnp.tanh(dis_ref[...] * tot + b_ref[...])
    gids = lax.broadcasted_iota(jnp.int32, (G, NP), 0)
    onehot = (batch_ref[...] == gids).astype(jnp.float32)
    sums = jnp.dot(onehot, t, preferred_element_type=jnp.float32)
    counts = jnp.sum(onehot, axis=1, keepdims=True)
    emb = sums / jnp.maximum(counts, 1.0)
    emb_ref[...] = emb
    out_ref[...] = (
        jnp.dot(emb, wlin_ref[...], preferred_element_type=jnp.float32)
        + blin_ref[...]
    )


_final_call = pl.pallas_call(
    _final_body,
    out_shape=(
        jax.ShapeDtypeStruct((G, 10), jnp.float32),
        jax.ShapeDtypeStruct((G, D), jnp.float32),
    ),
)


def kernel(x, edge_index, batch, W1, b1, W2, b2, W3, b3, W4, b4, W5, b5, Wlin, blin):
    src = edge_index[0].astype(jnp.int32)
    dst = edge_index[1].astype(jnp.int32)
    ppt = (EP - E) // NW
    padv = (N + jnp.arange(EP - E, dtype=jnp.int32) % (NP - N)).reshape(NW, ppt)
    src3 = jnp.concatenate(
        [src.reshape(NW, E // NW), padv], axis=1
    ).reshape(NW, NCHUNK, CHUNK)
    dst3 = jnp.concatenate(
        [dst.reshape(NW, E // NW), padv], axis=1
    ).reshape(NW, NCHUNK, CHUNK)
    x_pad = jnp.concatenate([x, jnp.zeros((NP - N, D), x.dtype)], axis=0)
    batch_pad = jnp.concatenate(
        [batch.astype(jnp.int32), jnp.full((NP - N,), G, jnp.int32)]
    ).reshape(1, NP)

    partials = _deg_call(dst3)
    dis, hp = _prep_call(partials.T, x_pad, W1)

    for W_next, b_prev in ((W2, b1), (W3, b2), (W4, b3), (W5, b4)):
        acc = _layer_call(hp, src3, dst3).reshape(2, NP, D)
        hp = _mid_call(acc, hp, dis, b_prev.reshape(1, D), W_next)

    acc = _layer_call(hp, src3, dst3).reshape(2, NP, D)
    out, emb = _final_call(
        acc, hp, dis, b5.reshape(1, D), batch_pad, Wlin, blin.reshape(1, -1)
    )
    return (out, emb)


# speedup vs baseline: 1.2674x; 1.0068x over previous
import jax
import jax.numpy as jnp
from jax import lax
from jax.experimental import pallas as pl
from jax.experimental.pallas import tpu as pltpu
from jax.experimental.pallas import tpu_sc as plsc

N = 10000
NP = 10240
D = 128
G = 128
E = 320000
NW = 32
CHUNK = 128
NCHUNK = 80
HALF = NCHUNK // 2
HGROUP = HALF // 2
EP = NW * NCHUNK * CHUNK
RPS = NP // 16

_mesh = plsc.VectorSubcoreMesh(
    core_axis_name="c", subcore_axis_name="s", num_cores=2, num_subcores=16
)


def _deg_body(dst_hbm, out_hbm, dstv, hist):
    c = lax.axis_index("c")
    s = lax.axis_index("s")
    wid = c * 16 + s
    pltpu.sync_copy(dst_hbm.at[wid], dstv)

    def zero_body(i, carry):
        hist[pl.ds(i * 16, 16)] = jnp.zeros((16,), jnp.float32)
        return carry

    lax.fori_loop(0, NP // 16, zero_body, 0)

    ones = jnp.ones((16,), jnp.float32)

    def hist_body(k, carry):
        kc = k // (CHUNK // 16)
        kj = k % (CHUNK // 16)
        idx = dstv[kc, pl.ds(kj * 16, 16)]
        plsc.addupdate_scatter(hist, [idx], ones)
        return carry

    lax.fori_loop(0, NCHUNK * (CHUNK // 16), hist_body, 0)
    pltpu.sync_copy(hist, out_hbm.at[wid])


_sc_params = pltpu.CompilerParams(needs_layout_passes=False)

_deg_call = pl.kernel(
    _deg_body,
    out_type=jax.ShapeDtypeStruct((NW, NP), jnp.float32),
    mesh=_mesh,
    compiler_params=_sc_params,
    scratch_types=[
        pltpu.VMEM((NCHUNK, CHUNK), jnp.int32),
        pltpu.VMEM((NP,), jnp.float32),
    ],
)


def _layer_body(
    hp_hbm, src_hbm, dst_hbm, out_hbm, srcv, dstv, rows0, rows1, acc,
    gsem0, gsem1,
):
    c = lax.axis_index("c")
    s = lax.axis_index("s")
    wid = c * 16 + s

    def zero_body(i, carry):
        rows0[i // (D // 16), pl.ds((i % (D // 16)) * 16, 16)] = jnp.zeros(
            (16,), jnp.float32
        )
        return carry

    lax.fori_loop(0, CHUNK * (D // 16), zero_body, 0)
    for r in range(RPS // CHUNK):
        pltpu.sync_copy(rows0, acc.at[pl.ds(s * RPS + r * CHUNK, CHUNK)])
    plsc.subcore_barrier()

    for phase in range(NCHUNK // HALF):
        base = phase * HALF
        pltpu.sync_copy(src_hbm.at[wid, pl.ds(base, HALF)], srcv)
        pltpu.sync_copy(dst_hbm.at[wid, pl.ds(base, HALF)], dstv)
        pltpu.async_copy(hp_hbm.at[srcv.at[0]], rows0, gsem0)

        def group_body(g, carry):
            j0 = 2 * g
            j1 = 2 * g + 1
            pltpu.async_copy(hp_hbm.at[srcv.at[j1]], rows1, gsem1)
            pltpu.make_async_copy(hp_hbm.at[srcv.at[j0]], rows0, gsem0).wait()
            pltpu.sync_copy(rows0, acc.at[dstv.at[j0]], add=True)

            @pl.when(g + 1 < HGROUP)
            def _():
                pltpu.async_copy(hp_hbm.at[srcv.at[j0 + 2]], rows0, gsem0)

            pltpu.make_async_copy(hp_hbm.at[srcv.at[j1]], rows1, gsem1).wait()
            pltpu.sync_copy(rows1, acc.at[dstv.at[j1]], add=True)
            return carry

        lax.fori_loop(0, HGROUP, group_body, 0)

    plsc.subcore_barrier()
    pltpu.sync_copy(
        acc.at[pl.ds(s * RPS, RPS)], out_hbm.at[pl.ds(c * NP + s * RPS, RPS)]
    )


_layer_call = pl.kernel(
    _layer_body,
    out_type=jax.ShapeDtypeStruct((2 * NP, D), jnp.float32),
    mesh=_mesh,
    compiler_params=_sc_params,
    scratch_types=[
        pltpu.VMEM((HALF, CHUNK), jnp.int32),
        pltpu.VMEM((HALF, CHUNK), jnp.int32),
        pltpu.VMEM((CHUNK, D), jnp.float32),
        pltpu.VMEM((CHUNK, D), jnp.float32),
        pltpu.VMEM_SHARED((NP, D), jnp.float32),
        pltpu.SemaphoreType.DMA,
        pltpu.SemaphoreType.DMA,
    ],
)


def _mm_body(x_ref, w1_ref, xw_ref):
    xw_ref[...] = jnp.dot(
        x_ref[...], w1_ref[...], preferred_element_type=jnp.float32
    )


_mm_call = pl.pallas_call(
    _mm_body,
    out_shape=jax.ShapeDtypeStruct((NP, D), jnp.float32),
)


def _prep_body(partials_ref, xw_ref, dis_ref, hp_ref):
    deg = 1.0 + jnp.sum(partials_ref[...], axis=1, keepdims=True)
    row = lax.broadcasted_iota(jnp.int32, (NP, 1), 0)
    dis = jnp.where(row < N, lax.rsqrt(deg), 0.0)
    dis_ref[...] = dis
    hp_ref[...] = dis * xw_ref[...]


_prep_call = pl.pallas_call(
    _prep_body,
    out_shape=(
        jax.ShapeDtypeStruct((NP, 1), jnp.float32),
        jax.ShapeDtypeStruct((NP, D), jnp.float32),
    ),
)


def _mid_body(acc_ref, hp_ref, dis_ref, b_ref, w_ref, out_ref):
    tot = acc_ref[0] + acc_ref[1] + hp_ref[...]
    t = jnp.tanh(dis_ref[...] * tot + b_ref[...])
    out_ref[...] = dis_ref[...] * jnp.dot(
        t, w_ref[...], preferred_element_type=jnp.float32
    )


_mid_call = pl.pallas_call(
    _mid_body,
    out_shape=jax.ShapeDtypeStruct((NP, D), jnp.float32),
)


def _final_body(
    acc_ref, hp_ref, dis_ref, b_ref, batch_ref, wlin_ref, blin_ref, out_ref, emb_ref
):
    tot = acc_ref[0] + acc_ref[1] + hp_ref[...]
    t = jnp.tanh(dis_ref[...] * tot + b_ref[...])
    gids = lax.broadcasted_iota(jnp.int32, (G, NP), 0)
    onehot = (batch_ref[...] == gids).astype(jnp.float32)
    sums = jnp.dot(onehot, t, preferred_element_type=jnp.float32)
    counts = jnp.sum(onehot, axis=1, keepdims=True)
    emb = sums / jnp.maximum(counts, 1.0)
    emb_ref[...] = emb
    out_ref[...] = (
        jnp.dot(emb, wlin_ref[...], preferred_element_type=jnp.float32)
        + blin_ref[...]
    )


_final_call = pl.pallas_call(
    _final_body,
    out_shape=(
        jax.ShapeDtypeStruct((G, 10), jnp.float32),
        jax.ShapeDtypeStruct((G, D), jnp.float32),
    ),
)


def kernel(x, edge_index, batch, W1, b1, W2, b2, W3, b3, W4, b4, W5, b5, Wlin, blin):
    src = edge_index[0].astype(jnp.int32)
    dst = edge_index[1].astype(jnp.int32)
    ppt = (EP - E) // NW
    padv = (N + jnp.arange(EP - E, dtype=jnp.int32) % (NP - N)).reshape(NW, ppt)
    src3 = jnp.concatenate(
        [src.reshape(NW, E // NW), padv], axis=1
    ).reshape(NW, NCHUNK, CHUNK)
    dst3 = jnp.concatenate(
        [dst.reshape(NW, E // NW), padv], axis=1
    ).reshape(NW, NCHUNK, CHUNK)
    x_pad = jnp.concatenate([x, jnp.zeros((NP - N, D), x.dtype)], axis=0)
    batch_pad = jnp.concatenate(
        [batch.astype(jnp.int32), jnp.full((NP - N,), G, jnp.int32)]
    ).reshape(1, NP)

    xw = _mm_call(x_pad, W1)
    partials = _deg_call(dst3)
    dis, hp = _prep_call(partials.T, xw)

    for W_next, b_prev in ((W2, b1), (W3, b2), (W4, b3), (W5, b4)):
        acc = _layer_call(hp, src3, dst3).reshape(2, NP, D)
        hp = _mid_call(acc, hp, dis, b_prev.reshape(1, D), W_next)

    acc = _layer_call(hp, src3, dst3).reshape(2, NP, D)
    out, emb = _final_call(
        acc, hp, dis, b5.reshape(1, D), batch_pad, Wlin, blin.reshape(1, -1)
    )
    return (out, emb)


# speedup vs baseline: 1.2688x; 1.0011x over previous
import jax
import jax.numpy as jnp
from jax import lax
from jax.experimental import pallas as pl
from jax.experimental.pallas import tpu as pltpu
from jax.experimental.pallas import tpu_sc as plsc

N = 10000
NP = 10240
D = 128
G = 128
E = 320000
NW = 32
CHUNK = 128
NCHUNK = 80
HALF = NCHUNK // 2
HGROUP = HALF // 2
EP = NW * NCHUNK * CHUNK
RPS = NP // 16

_mesh = plsc.VectorSubcoreMesh(
    core_axis_name="c", subcore_axis_name="s", num_cores=2, num_subcores=16
)


def _deg_body(dst_hbm, out_hbm, dstv, hist):
    c = lax.axis_index("c")
    s = lax.axis_index("s")
    wid = c * 16 + s
    pltpu.sync_copy(dst_hbm.at[wid], dstv)

    def zero_body(i, carry):
        hist[pl.ds(i * 16, 16)] = jnp.zeros((16,), jnp.float32)
        return carry

    lax.fori_loop(0, NP // 16, zero_body, 0)

    ones = jnp.ones((16,), jnp.float32)

    def hist_body(k, carry):
        kc = k // (CHUNK // 16)
        kj = k % (CHUNK // 16)
        idx = dstv[kc, pl.ds(kj * 16, 16)]
        plsc.addupdate_scatter(hist, [idx], ones)
        return carry

    lax.fori_loop(0, NCHUNK * (CHUNK // 16), hist_body, 0)
    pltpu.sync_copy(hist, out_hbm.at[wid])


_sc_params = pltpu.CompilerParams(needs_layout_passes=False)

_deg_call = pl.kernel(
    _deg_body,
    out_type=jax.ShapeDtypeStruct((NW, NP), jnp.float32),
    mesh=_mesh,
    compiler_params=_sc_params,
    scratch_types=[
        pltpu.VMEM((NCHUNK, CHUNK), jnp.int32),
        pltpu.VMEM((NP,), jnp.float32),
    ],
)


def _layer_body(
    hp_hbm, src_hbm, dst_hbm, out_hbm, srcv, dstv, rows0, rows1, acc,
    gsem0, gsem1,
):
    c = lax.axis_index("c")
    s = lax.axis_index("s")
    wid = c * 16 + s

    def zero_body(i, carry):
        rows0[i // (D // 16), pl.ds((i % (D // 16)) * 16, 16)] = jnp.zeros(
            (16,), jnp.float32
        )
        return carry

    lax.fori_loop(0, CHUNK * (D // 16), zero_body, 0)
    for r in range(RPS // CHUNK):
        pltpu.sync_copy(rows0, acc.at[pl.ds(s * RPS + r * CHUNK, CHUNK)])
    plsc.subcore_barrier()

    for phase in range(NCHUNK // HALF):
        base = phase * HALF
        pltpu.sync_copy(src_hbm.at[wid, pl.ds(base, HALF)], srcv)
        pltpu.sync_copy(dst_hbm.at[wid, pl.ds(base, HALF)], dstv)
        pltpu.async_copy(hp_hbm.at[srcv.at[0]], rows0, gsem0)

        def group_body(g, carry):
            j0 = 4 * g
            for u in range(0, 4, 2):
                ja = j0 + u
                jb = j0 + u + 1
                pltpu.async_copy(hp_hbm.at[srcv.at[jb]], rows1, gsem1)
                pltpu.make_async_copy(
                    hp_hbm.at[srcv.at[ja]], rows0, gsem0
                ).wait()
                pltpu.sync_copy(rows0, acc.at[dstv.at[ja]], add=True)

                @pl.when(ja + 2 < HALF)
                def _():
                    pltpu.async_copy(hp_hbm.at[srcv.at[ja + 2]], rows0, gsem0)

                pltpu.make_async_copy(
                    hp_hbm.at[srcv.at[jb]], rows1, gsem1
                ).wait()
                pltpu.sync_copy(rows1, acc.at[dstv.at[jb]], add=True)
            return carry

        lax.fori_loop(0, HALF // 4, group_body, 0)

    plsc.subcore_barrier()
    pltpu.sync_copy(
        acc.at[pl.ds(s * RPS, RPS)], out_hbm.at[pl.ds(c * NP + s * RPS, RPS)]
    )


_layer_call = pl.kernel(
    _layer_body,
    out_type=jax.ShapeDtypeStruct((2 * NP, D), jnp.float32),
    mesh=_mesh,
    compiler_params=_sc_params,
    scratch_types=[
        pltpu.VMEM((HALF, CHUNK), jnp.int32),
        pltpu.VMEM((HALF, CHUNK), jnp.int32),
        pltpu.VMEM((CHUNK, D), jnp.float32),
        pltpu.VMEM((CHUNK, D), jnp.float32),
        pltpu.VMEM_SHARED((NP, D), jnp.float32),
        pltpu.SemaphoreType.DMA,
        pltpu.SemaphoreType.DMA,
    ],
)


def _mm_body(x_ref, w1_ref, xw_ref):
    xw_ref[...] = jnp.dot(
        x_ref[...], w1_ref[...], preferred_element_type=jnp.float32
    )


_mm_call = pl.pallas_call(
    _mm_body,
    out_shape=jax.ShapeDtypeStruct((NP, D), jnp.float32),
)


def _prep_body(partials_ref, xw_ref, dis_ref, hp_ref):
    deg = 1.0 + jnp.sum(partials_ref[...], axis=1, keepdims=True)
    row = lax.broadcasted_iota(jnp.int32, (NP, 1), 0)
    dis = jnp.where(row < N, lax.rsqrt(deg), 0.0)
    dis_ref[...] = dis
    hp_ref[...] = dis * xw_ref[...]


_prep_call = pl.pallas_call(
    _prep_body,
    out_shape=(
        jax.ShapeDtypeStruct((NP, 1), jnp.float32),
        jax.ShapeDtypeStruct((NP, D), jnp.float32),
    ),
)


def _mid_body(acc_ref, hp_ref, dis_ref, b_ref, w_ref, out_ref):
    tot = acc_ref[0] + acc_ref[1] + hp_ref[...]
    t = jnp.tanh(dis_ref[...] * tot + b_ref[...])
    out_ref[...] = dis_ref[...] * jnp.dot(
        t, w_ref[...], preferred_element_type=jnp.float32
    )


_mid_call = pl.pallas_call(
    _mid_body,
    out_shape=jax.ShapeDtypeStruct((NP, D), jnp.float32),
)


def _final_body(
    acc_ref, hp_ref, dis_ref, b_ref, batch_ref, wlin_ref, blin_ref, out_ref, emb_ref
):
    tot = acc_ref[0] + acc_ref[1] + hp_ref[...]
    t = jnp.tanh(dis_ref[...] * tot + b_ref[...])
    gids = lax.broadcasted_iota(jnp.int32, (G, NP), 0)
    onehot = (batch_ref[...] == gids).astype(jnp.float32)
    sums = jnp.dot(onehot, t, preferred_element_type=jnp.float32)
    counts = jnp.sum(onehot, axis=1, keepdims=True)
    emb = sums / jnp.maximum(counts, 1.0)
    emb_ref[...] = emb
    out_ref[...] = (
        jnp.dot(emb, wlin_ref[...], preferred_element_type=jnp.float32)
        + blin_ref[...]
    )


_final_call = pl.pallas_call(
    _final_body,
    out_shape=(
        jax.ShapeDtypeStruct((G, 10), jnp.float32),
        jax.ShapeDtypeStruct((G, D), jnp.float32),
    ),
)


def kernel(x, edge_index, batch, W1, b1, W2, b2, W3, b3, W4, b4, W5, b5, Wlin, blin):
    src = edge_index[0].astype(jnp.int32)
    dst = edge_index[1].astype(jnp.int32)
    ppt = (EP - E) // NW
    padv = (N + jnp.arange(EP - E, dtype=jnp.int32) % (NP - N)).reshape(NW, ppt)
    src3 = jnp.concatenate(
        [src.reshape(NW, E // NW), padv], axis=1
    ).reshape(NW, NCHUNK, CHUNK)
    dst3 = jnp.concatenate(
        [dst.reshape(NW, E // NW), padv], axis=1
    ).reshape(NW, NCHUNK, CHUNK)
    x_pad = jnp.concatenate([x, jnp.zeros((NP - N, D), x.dtype)], axis=0)
    batch_pad = jnp.concatenate(
        [batch.astype(jnp.int32), jnp.full((NP - N,), G, jnp.int32)]
    ).reshape(1, NP)

    xw = _mm_call(x_pad, W1)
    partials = _deg_call(dst3)
    dis, hp = _prep_call(partials.T, xw)

    for W_next, b_prev in ((W2, b1), (W3, b2), (W4, b3), (W5, b4)):
        acc = _layer_call(hp, src3, dst3).reshape(2, NP, D)
        hp = _mid_call(acc, hp, dis, b_prev.reshape(1, D), W_next)

    acc = _layer_call(hp, src3, dst3).reshape(2, NP, D)
    out, emb = _final_call(
        acc, hp, dis, b5.reshape(1, D), batch_pad, Wlin, blin.reshape(1, -1)
    )
    return (out, emb)
